# Initial kernel scaffold; baseline (speedup 1.0000x reference)
import jax
import jax.numpy as jnp
from jax.experimental import pallas as pl


def kernel(title_ids, text_token_ids, title_table, text_table):
    raise NotImplementedError("write your pallas kernel here")


# speedup vs baseline: 11.2086x; 11.2086x over previous
import functools

import jax
import jax.numpy as jnp
from jax import lax
from jax.experimental import pallas as pl
from jax.experimental.pallas import tpu as pltpu
from jax.experimental.pallas import tpu_sc as plsc

B = 16384
SEQ = 20
D = 32
NW = 32
BPW = B // NW
GW = 128
CH = 64
NCHUNK = BPW // CH
ROWS_PER_CHUNK = CH * SEQ
GATHERS_PER_CHUNK = ROWS_PER_CHUNK // GW


def _sc_embed(title_table, title_ids, text_table_z, text_idx_flat):
    mesh = plsc.VectorSubcoreMesh(core_axis_name="c", subcore_axis_name="s")

    @functools.partial(
        pl.kernel,
        out_type=(
            jax.ShapeDtypeStruct((B, D), jnp.float32),
            jax.ShapeDtypeStruct((B, D), jnp.float32),
        ),
        mesh=mesh,
        compiler_params=pltpu.CompilerParams(use_tc_tiling_on_sc=False),
        scratch_types=[
            pltpu.VMEM((BPW,), jnp.int32),
            pltpu.VMEM((BPW, D), jnp.float32),
            pltpu.VMEM((BPW * SEQ,), jnp.int32),
            pltpu.VMEM((ROWS_PER_CHUNK, D), jnp.float32),
            pltpu.VMEM((BPW, D), jnp.float32),
            pltpu.SemaphoreType.DMA,
        ],
    )
    def sc_kernel(title_tab, title_idx, text_tab, text_idx,
                  out_title, out_sums,
                  tidx_v, trows_v, xidx_v, xrows_v, acc_v, sem):
        wid = lax.axis_index("s") * 2 + lax.axis_index("c")
        base = wid * BPW

        pltpu.sync_copy(title_idx.at[pl.ds(base, BPW)], tidx_v)
        pltpu.sync_copy(text_idx.at[pl.ds(base * SEQ, BPW * SEQ)], xidx_v)

        cps = [
            pltpu.async_copy(
                title_tab.at[tidx_v.at[pl.ds(j * GW, GW)]],
                trows_v.at[pl.ds(j * GW, GW)],
                sem,
            )
            for j in range(BPW // GW)
        ]
        for cp in cps:
            cp.wait()
        pltpu.sync_copy(trows_v, out_title.at[pl.ds(base, BPW)])

        @pl.loop(0, NCHUNK)
        def _chunk(c):
            gcps = [
                pltpu.async_copy(
                    text_tab.at[xidx_v.at[pl.ds(c * ROWS_PER_CHUNK + j * GW, GW)]],
                    xrows_v.at[pl.ds(j * GW, GW)],
                    sem,
                )
                for j in range(GATHERS_PER_CHUNK)
            ]
            for cp in gcps:
                cp.wait()

            @pl.loop(0, CH)
            def _sample(s):
                r0 = s * SEQ
                for h in range(D // 16):
                    col = pl.ds(16 * h, 16)
                    acc = xrows_v[r0, col]
                    for j in range(1, SEQ):
                        acc = acc + xrows_v[r0 + j, col]
                    acc_v[c * CH + s, col] = acc

        pltpu.sync_copy(acc_v, out_sums.at[pl.ds(base, BPW)])

    return sc_kernel(title_table, title_ids, text_table_z, text_idx_flat)


def _tc_finish(title_emb, sums, text_token_ids):
    BLK = 2048

    def body(ids_ref, title_ref, sums_ref, o_ref):
        ids = ids_ref[...]
        n = jnp.sum((ids != 0).astype(jnp.float32), axis=1, keepdims=True)
        text = sums_ref[...] / jnp.maximum(n, 1.0)
        o_ref[...] = jnp.concatenate([title_ref[...], text], axis=1)

    return pl.pallas_call(
        body,
        grid=(B // BLK,),
        in_specs=[
            pl.BlockSpec((BLK, SEQ), lambda i: (i, 0)),
            pl.BlockSpec((BLK, D), lambda i: (i, 0)),
            pl.BlockSpec((BLK, D), lambda i: (i, 0)),
        ],
        out_specs=pl.BlockSpec((BLK, 2 * D), lambda i: (i, 0)),
        out_shape=jax.ShapeDtypeStruct((B, 2 * D), jnp.float32),
    )(text_token_ids, title_emb, sums)


def kernel(title_ids, text_token_ids, title_table, text_table):
    text_table_z = text_table.at[0].set(0.0)
    text_idx_flat = text_token_ids.reshape(-1)
    title_emb, sums = _sc_embed(title_table, title_ids, text_table_z, text_idx_flat)
    return _tc_finish(title_emb, sums, text_token_ids)


# speedup vs baseline: 14.2100x; 1.2678x over previous
import functools

import jax
import jax.numpy as jnp
from jax import lax
from jax.experimental import pallas as pl
from jax.experimental.pallas import tpu as pltpu
from jax.experimental.pallas import tpu_sc as plsc

B = 16384
SEQ = 20
D = 32
NW = 32
BPW = B // NW
GW = 128
CH = 64
NCHUNK = BPW // CH
ROWS_PER_CHUNK = CH * SEQ
GATHERS_PER_CHUNK = ROWS_PER_CHUNK // GW

_MESH = plsc.VectorSubcoreMesh(core_axis_name="c", subcore_axis_name="s")
_NOTILE = pltpu.CompilerParams(use_tc_tiling_on_sc=False)
if "needs_layout_passes" in pltpu.CompilerParams.__dataclass_fields__:
    import dataclasses as _dc
    _NOTILE = _dc.replace(_NOTILE, needs_layout_passes=False)


def _worker_base(samples_per_worker):
    wid = lax.axis_index("s") * 2 + lax.axis_index("c")
    return wid * samples_per_worker


def _sc_title(title_table, title_ids):
    @functools.partial(
        pl.kernel,
        out_type=jax.ShapeDtypeStruct((B, D), jnp.float32),
        mesh=_MESH,
        compiler_params=_NOTILE,
        scratch_types=[
            pltpu.VMEM((BPW,), jnp.int32),
            pltpu.VMEM((BPW, D), jnp.float32),
            pltpu.SemaphoreType.DMA,
        ],
    )
    def sc_kernel(tab, idx, out, idx_v, rows_v, sem):
        base = _worker_base(BPW)
        pltpu.sync_copy(idx.at[pl.ds(base, BPW)], idx_v)
        cps = [
            pltpu.async_copy(
                tab.at[idx_v.at[pl.ds(j * GW, GW)]],
                rows_v.at[pl.ds(j * GW, GW)],
                sem,
            )
            for j in range(BPW // GW)
        ]
        for cp in cps:
            cp.wait()
        pltpu.sync_copy(rows_v, out.at[pl.ds(base, BPW)])

    return sc_kernel(title_table, title_ids)


def _sc_text(text_table, text_idx_flat):
    @functools.partial(
        pl.kernel,
        out_type=jax.ShapeDtypeStruct((B, D), jnp.float32),
        mesh=_MESH,
        compiler_params=_NOTILE,
        scratch_types=[
            pltpu.VMEM((BPW * SEQ,), jnp.int32),
            pltpu.VMEM((ROWS_PER_CHUNK, D), jnp.float32),
            pltpu.VMEM((BPW, D), jnp.float32),
            pltpu.VMEM((1, D), jnp.float32),
            pltpu.SemaphoreType.DMA,
        ],
    )
    def sc_kernel(tab, idx, out, idx_v, rows_v, tout_v, row0_v, sem):
        base = _worker_base(BPW)
        pltpu.sync_copy(idx.at[pl.ds(base * SEQ, BPW * SEQ)], idx_v)
        pltpu.sync_copy(tab.at[pl.ds(0, 1)], row0_v)
        lane = lax.iota(jnp.int32, 16)

        @pl.loop(0, NCHUNK)
        def _chunk(c):
            gcps = [
                pltpu.async_copy(
                    tab.at[idx_v.at[pl.ds(c * ROWS_PER_CHUNK + j * GW, GW)]],
                    rows_v.at[pl.ds(j * GW, GW)],
                    sem,
                )
                for j in range(GATHERS_PER_CHUNK)
            ]
            for cp in gcps:
                cp.wait()

            @pl.loop(0, CH)
            def _sample(s):
                r0 = s * SEQ
                v1 = idx_v[pl.ds(c * ROWS_PER_CHUNK + r0, 16)]
                v2 = idx_v[pl.ds(c * ROWS_PER_CHUNK + r0 + 4, 16)]
                z1 = v1 == 0
                z2 = jnp.logical_and(v2 == 0, lane >= 12)
                npad = (plsc.all_reduce_population_count(z1)
                        + plsc.all_reduce_population_count(z2))
                npad_f = npad.astype(jnp.float32)
                inv = 1.0 / jnp.maximum(20.0 - npad_f, 1.0)
                for h in range(D // 16):
                    col = pl.ds(16 * h, 16)
                    acc = rows_v[r0, col]
                    for j in range(1, SEQ):
                        acc = acc + rows_v[r0 + j, col]
                    tout_v[c * CH + s, col] = (acc - npad_f * row0_v[0, col]) * inv

        pltpu.sync_copy(tout_v, out.at[pl.ds(base, BPW)])

    return sc_kernel(text_table, text_idx_flat)


def kernel(title_ids, text_token_ids, title_table, text_table):
    title_emb = _sc_title(title_table, title_ids)
    text_emb = _sc_text(text_table, text_token_ids.reshape(-1))
    return jnp.concatenate([title_emb, text_emb], axis=1)
